# baseline (speedup 1.0000x reference)
import functools

import jax
import jax.numpy as jnp
from jax import lax
from jax.experimental import pallas as pl
from jax.experimental.pallas import tpu as pltpu
from jax.experimental.pallas import tpu_sc as plsc

N = 10000
E = 320000
D = 128
CH = 128
NS = 16
NC = 2

N_PAD = 10240
E2 = E // 2
CPS = 1280
EPAD = CPS * CH - E2
CPT = CPS // NS
RPT = N_PAD // NS
G = 16
DG = 16
RB = 1024

_MESH = plsc.VectorSubcoreMesh(core_axis_name="c", subcore_axis_name="s")
_SC_LINEAR = pltpu.CompilerParams(use_tc_tiling_on_sc=False)
_SC_TILED = pltpu.CompilerParams(use_tc_tiling_on_sc=True)


@functools.partial(
    pl.kernel,
    out_type=[jax.ShapeDtypeStruct((N_PAD, 8), jnp.float32),
              jax.ShapeDtypeStruct((N_PAD, 8), jnp.float32)],
    mesh=_MESH,
    compiler_params=_SC_LINEAR,
    scratch_types=[
        pltpu.VMEM_SHARED((N_PAD, 8), jnp.float32),
        pltpu.VMEM((2 * CPT, CH), jnp.int32),
        pltpu.VMEM((CH, 8), jnp.float32),
        pltpu.SemaphoreType.DMA,
    ],
)
def _deg_kernel(e_hbm, ones_hbm, zeros_hbm, deg0_hbm, deg1_hbm, deg_spmem,
                idx_v, ones_v, sem):
    c = lax.axis_index("c")
    s = lax.axis_index("s")

    pltpu.sync_copy(ones_hbm, ones_v)
    pltpu.sync_copy(zeros_hbm.at[pl.ds(s * RPT, RPT)],
                    deg_spmem.at[pl.ds(s * RPT, RPT)])

    pltpu.sync_copy(e_hbm.at[pl.ds(2 * CPS * c + s * (2 * CPT), 2 * CPT)],
                    idx_v)

    plsc.subcore_barrier()

    def _count_group(g, carry):
        descs = [
            pltpu.async_copy(ones_v, deg_spmem.at[idx_v.at[g * DG + k]], sem,
                             add=True)
            for k in range(DG)
        ]
        for d in descs:
            d.wait()
        return carry

    lax.fori_loop(0, (2 * CPT) // DG, _count_group, 0)

    plsc.subcore_barrier()

    @pl.when(c == 0)
    def _():
        pltpu.sync_copy(deg_spmem.at[pl.ds(s * RPT, RPT)],
                        deg0_hbm.at[pl.ds(s * RPT, RPT)])

    @pl.when(c == 1)
    def _():
        pltpu.sync_copy(deg_spmem.at[pl.ds(s * RPT, RPT)],
                        deg1_hbm.at[pl.ds(s * RPT, RPT)])


@functools.partial(
    pl.kernel,
    out_type=[jax.ShapeDtypeStruct((N_PAD, D), jnp.float32),
              jax.ShapeDtypeStruct((N_PAD, D), jnp.float32)],
    mesh=_MESH,
    compiler_params=_SC_TILED,
    scratch_types=[
        pltpu.VMEM_SHARED((N_PAD, D), jnp.float32),
        pltpu.VMEM((G, CH), jnp.int32),
        pltpu.VMEM((G, CH), jnp.int32),
        pltpu.VMEM((G, CH), jnp.int32),
        pltpu.VMEM((G, CH), jnp.int32),
        pltpu.VMEM((CH, D), jnp.float32),
        pltpu.VMEM((CH, D), jnp.float32),
        pltpu.SemaphoreType.DMA,
        pltpu.SemaphoreType.DMA,
        pltpu.SemaphoreType.DMA,
        pltpu.SemaphoreType.DMA,
        pltpu.SemaphoreType.DMA,
        pltpu.SemaphoreType.DMA,
        pltpu.SemaphoreType.DMA,
        pltpu.SemaphoreType.DMA,
    ],
)
def _agg_kernel(y_hbm, e_hbm, agg0_hbm, agg1_hbm, agg_spmem, sidx0_v, didx0_v,
                sidx1_v, didx1_v, msg_a, msg_b, gsem_l, gsem_h, ssem_a, ssem_b,
                isem_s0, isem_d0, isem_s1, isem_d1):
    c = lax.axis_index("c")
    s = lax.axis_index("s")

    def _fill_zeros(i, carry):
        for k in range(D // 16):
            msg_a[i, pl.ds(k * 16, 16)] = jnp.zeros((16,), jnp.float32)
        return carry

    lax.fori_loop(0, CH, _fill_zeros, 0)
    for z in range(RPT // CH):
        pltpu.sync_copy(msg_a, agg_spmem.at[pl.ds(s * RPT + z * CH, CH)])

    plsc.subcore_barrier()

    sbase = CPS * c + s * CPT
    dbase = 2 * CPS + CPS * c + s * CPT

    bufs = (msg_a, msg_b)
    ssems = (ssem_a, ssem_b)
    sidxs = (sidx0_v, sidx1_v)
    didxs = (didx0_v, didx1_v)
    isems_s = (isem_s0, isem_s1)
    isems_d = (isem_d0, isem_d1)

    def _prefetch(g, par):
        pltpu.async_copy(e_hbm.at[pl.ds(sbase + g * G, G)], sidxs[par],
                         isems_s[par])
        pltpu.async_copy(e_hbm.at[pl.ds(dbase + g * G, G)], didxs[par],
                         isems_d[par])

    def _wait_idx(g, par):
        pltpu.make_async_copy(e_hbm.at[pl.ds(sbase + g * G, G)], sidxs[par],
                              isems_s[par]).wait()
        pltpu.make_async_copy(e_hbm.at[pl.ds(dbase + g * G, G)], didxs[par],
                              isems_d[par]).wait()

    def _work(g, par):
        _wait_idx(g, par)
        sidx_v = sidxs[par]
        didx_v = didxs[par]

        def _gather2(k, buf):
            q = CH // 4
            descs = []
            for i in range(4):
                descs.append(pltpu.async_copy(
                    y_hbm.at[sidx_v.at[k, pl.ds(i * q, q)]],
                    buf.at[pl.ds(i * q, q)],
                    gsem_l if i % 2 == 0 else gsem_h))
            return descs

        gd = [None] * G
        sd = [None] * G
        gd[0] = _gather2(0, bufs[0])
        for k in range(G):
            p = k % 2
            for d in gd[k]:
                d.wait()
            if k + 1 < G:
                if k >= 1:
                    sd[k - 1].wait()
                gd[k + 1] = _gather2(k + 1, bufs[1 - p])
            sd[k] = pltpu.async_copy(bufs[p], agg_spmem.at[didx_v.at[k]],
                                     ssems[p], add=True)
        sd[G - 2].wait()
        sd[G - 1].wait()

    NGRP = CPT // G
    _prefetch(0, 0)
    _prefetch(1, 1)

    def _pair(t, carry):
        ga = 2 * t
        _work(ga, 0)
        _prefetch(ga + 2, 0)
        _work(ga + 1, 1)

        @pl.when(t == 0)
        def _():
            _prefetch(ga + 3, 1)

        return carry

    lax.fori_loop(0, (NGRP - 1) // 2, _pair, 0)
    _work(NGRP - 1, 0)

    plsc.subcore_barrier()

    @pl.when(c == 0)
    def _():
        pltpu.sync_copy(agg_spmem.at[pl.ds(s * RPT, RPT)],
                        agg0_hbm.at[pl.ds(s * RPT, RPT)])

    @pl.when(c == 1)
    def _():
        pltpu.sync_copy(agg_spmem.at[pl.ds(s * RPT, RPT)],
                        agg1_hbm.at[pl.ds(s * RPT, RPT)])


def _tc_matmul(xp, W):
    def body(x_ref, w_ref, y_ref):
        y_ref[...] = jnp.dot(x_ref[...], w_ref[...],
                             preferred_element_type=jnp.float32)

    return pl.pallas_call(
        body,
        grid=(N_PAD // RB,),
        in_specs=[
            pl.BlockSpec((RB, D), lambda i: (i, 0)),
            pl.BlockSpec((D, D), lambda i: (0, 0)),
        ],
        out_specs=pl.BlockSpec((RB, D), lambda i: (i, 0)),
        out_shape=jax.ShapeDtypeStruct((N_PAD, D), jnp.float32),
    )(xp, W)


def _tc_scale(y0, deg_src):
    def body(y_ref, deg_ref, o_ref):
        scale = lax.rsqrt(jnp.maximum(deg_ref[:, 0:1], 1.0))
        o_ref[...] = y_ref[...] * scale

    return pl.pallas_call(
        body,
        grid=(N_PAD // RB,),
        in_specs=[
            pl.BlockSpec((RB, D), lambda i: (i, 0)),
            pl.BlockSpec((RB, 8), lambda i: (i, 0)),
        ],
        out_specs=pl.BlockSpec((RB, D), lambda i: (i, 0)),
        out_shape=jax.ShapeDtypeStruct((N_PAD, D), jnp.float32),
    )(y0, deg_src)


def _tc_post(agg0, agg1, deg_dst, b2):
    def body(a0_ref, a1_ref, deg_ref, b_ref, o_ref):
        scale = lax.rsqrt(jnp.maximum(deg_ref[:, 0:1], 1.0))
        o_ref[...] = (a0_ref[...] + a1_ref[...]) * scale + b_ref[0:1, :]

    rb = 1000
    return pl.pallas_call(
        body,
        grid=(N // rb,),
        in_specs=[
            pl.BlockSpec((rb, D), lambda i: (i, 0)),
            pl.BlockSpec((rb, D), lambda i: (i, 0)),
            pl.BlockSpec((rb, 8), lambda i: (i, 0)),
            pl.BlockSpec((1, D), lambda i: (0, 0)),
        ],
        out_specs=pl.BlockSpec((rb, D), lambda i: (i, 0)),
        out_shape=jax.ShapeDtypeStruct((N, D), jnp.float32),
    )(agg0, agg1, deg_dst, b2)


def kernel(x, edge_index, W, b):
    ep = jnp.pad(edge_index, ((0, 0), (0, 2 * CPS * CH - E)),
                 constant_values=N)
    e_all = ep.reshape(4 * CPS, CH)
    xp = jnp.pad(x, ((0, N_PAD - N), (0, 0)))

    deg_src, deg_dst = _deg_kernel(e_all, jnp.ones((CH, 8), jnp.float32),
                                   jnp.zeros((N_PAD, 8), jnp.float32))
    y0 = _tc_matmul(xp, W)
    y = _tc_scale(y0, deg_src)
    agg0, agg1 = _agg_kernel(y, e_all)
    return _tc_post(agg0, agg1, deg_dst, b.reshape(1, D))


# speedup vs baseline: 3.1728x; 3.1728x over previous
import functools

import jax
import jax.numpy as jnp
from jax import lax
from jax.experimental import pallas as pl
from jax.experimental.pallas import tpu as pltpu
from jax.experimental.pallas import tpu_sc as plsc

N = 10000
E = 320000
D = 128
CH = 128
NS = 16
NC = 2

N_PAD = 10240
E2 = E // 2
CPS = 1280
EPAD = CPS * CH - E2
CPT = CPS // NS
RPT = N_PAD // NS
G = 16
DG = 16
RB = 1024

_MESH = plsc.VectorSubcoreMesh(core_axis_name="c", subcore_axis_name="s")
_SC_LINEAR = pltpu.CompilerParams(use_tc_tiling_on_sc=False)
_SC_TILED = pltpu.CompilerParams(use_tc_tiling_on_sc=True)


@functools.partial(
    pl.kernel,
    out_type=[jax.ShapeDtypeStruct((N_PAD, 8), jnp.float32),
              jax.ShapeDtypeStruct((N_PAD, 8), jnp.float32)],
    mesh=_MESH,
    compiler_params=_SC_LINEAR,
    scratch_types=[
        pltpu.VMEM_SHARED((N_PAD, 8), jnp.float32),
        pltpu.VMEM((2 * CPT, CH), jnp.int32),
        pltpu.VMEM((CH, 8), jnp.float32),
        pltpu.SemaphoreType.DMA,
    ],
)
def _deg_kernel(e_hbm, ones_hbm, zeros_hbm, deg0_hbm, deg1_hbm, deg_spmem,
                idx_v, ones_v, sem):
    c = lax.axis_index("c")
    s = lax.axis_index("s")

    pltpu.sync_copy(ones_hbm, ones_v)
    pltpu.sync_copy(zeros_hbm.at[pl.ds(s * RPT, RPT)],
                    deg_spmem.at[pl.ds(s * RPT, RPT)])

    pltpu.sync_copy(e_hbm.at[pl.ds(2 * CPS * c + s * (2 * CPT), 2 * CPT)],
                    idx_v)

    plsc.subcore_barrier()

    def _count_group(g, carry):
        descs = [
            pltpu.async_copy(ones_v, deg_spmem.at[idx_v.at[g * DG + k]], sem,
                             add=True)
            for k in range(DG)
        ]
        for d in descs:
            d.wait()
        return carry

    lax.fori_loop(0, (2 * CPT) // DG, _count_group, 0)

    plsc.subcore_barrier()

    @pl.when(c == 0)
    def _():
        pltpu.sync_copy(deg_spmem.at[pl.ds(s * RPT, RPT)],
                        deg0_hbm.at[pl.ds(s * RPT, RPT)])

    @pl.when(c == 1)
    def _():
        pltpu.sync_copy(deg_spmem.at[pl.ds(s * RPT, RPT)],
                        deg1_hbm.at[pl.ds(s * RPT, RPT)])


@functools.partial(
    pl.kernel,
    out_type=[jax.ShapeDtypeStruct((N_PAD, D), jnp.float32),
              jax.ShapeDtypeStruct((N_PAD, D), jnp.float32)],
    mesh=_MESH,
    compiler_params=_SC_TILED,
    scratch_types=[
        pltpu.VMEM_SHARED((N_PAD, D), jnp.float32),
        pltpu.VMEM((G, CH), jnp.int32),
        pltpu.VMEM((G, CH), jnp.int32),
        pltpu.VMEM((G, CH), jnp.int32),
        pltpu.VMEM((G, CH), jnp.int32),
        pltpu.VMEM((CH, D), jnp.float32),
        pltpu.VMEM((CH, D), jnp.float32),
        pltpu.SemaphoreType.DMA,
        pltpu.SemaphoreType.DMA,
        pltpu.SemaphoreType.DMA,
        pltpu.SemaphoreType.DMA,
        pltpu.SemaphoreType.DMA,
        pltpu.SemaphoreType.DMA,
        pltpu.SemaphoreType.DMA,
        pltpu.SemaphoreType.DMA,
    ],
)
def _agg_kernel(y_hbm, e_hbm, agg0_hbm, agg1_hbm, agg_spmem, sidx0_v, didx0_v,
                sidx1_v, didx1_v, msg_a, msg_b, gsem_l, gsem_h, ssem_a, ssem_b,
                isem_s0, isem_d0, isem_s1, isem_d1):
    c = lax.axis_index("c")
    s = lax.axis_index("s")

    def _fill_zeros(i, carry):
        for k in range(D // 16):
            msg_a[i, pl.ds(k * 16, 16)] = jnp.zeros((16,), jnp.float32)
        return carry

    lax.fori_loop(0, CH, _fill_zeros, 0)
    for z in range(RPT // CH):
        pltpu.sync_copy(msg_a, agg_spmem.at[pl.ds(s * RPT + z * CH, CH)])

    plsc.subcore_barrier()

    sbase = CPS * c + s * CPT
    dbase = 2 * CPS + CPS * c + s * CPT

    bufs = (msg_a, msg_b)
    ssems = (ssem_a, ssem_b)
    sidxs = (sidx0_v, sidx1_v)
    didxs = (didx0_v, didx1_v)
    isems_s = (isem_s0, isem_s1)
    isems_d = (isem_d0, isem_d1)

    def _prefetch(g, par):
        pltpu.async_copy(e_hbm.at[pl.ds(sbase + g * G, G)], sidxs[par],
                         isems_s[par])
        pltpu.async_copy(e_hbm.at[pl.ds(dbase + g * G, G)], didxs[par],
                         isems_d[par])

    def _wait_idx(g, par):
        pltpu.make_async_copy(e_hbm.at[pl.ds(sbase + g * G, G)], sidxs[par],
                              isems_s[par]).wait()
        pltpu.make_async_copy(e_hbm.at[pl.ds(dbase + g * G, G)], didxs[par],
                              isems_d[par]).wait()

    def _work(g, par):
        _wait_idx(g, par)
        sidx_v = sidxs[par]
        didx_v = didxs[par]

        def _gather2(k, buf):
            q = CH // 4
            descs = []
            for i in range(4):
                descs.append(pltpu.async_copy(
                    y_hbm.at[sidx_v.at[k, pl.ds(i * q, q)]],
                    buf.at[pl.ds(i * q, q)],
                    gsem_l if i % 2 == 0 else gsem_h))
            return descs

        gd = [None] * G
        sd = [None] * G
        gd[0] = _gather2(0, bufs[0])
        for k in range(G):
            p = k % 2
            for d in gd[k]:
                d.wait()
            if k + 1 < G:
                if k >= 1:
                    sd[k - 1].wait()
                gd[k + 1] = _gather2(k + 1, bufs[1 - p])
            sd[k] = pltpu.async_copy(bufs[p], agg_spmem.at[didx_v.at[k]],
                                     ssems[p], add=True)
        sd[G - 2].wait()
        sd[G - 1].wait()

    NGRP = CPT // G
    _prefetch(0, 0)
    _prefetch(1, 1)

    def _pair(t, carry):
        ga = 2 * t
        _work(ga, 0)
        _prefetch(ga + 2, 0)
        _work(ga + 1, 1)

        @pl.when(t == 0)
        def _():
            _prefetch(ga + 3, 1)

        return carry

    lax.fori_loop(0, (NGRP - 1) // 2, _pair, 0)
    _work(NGRP - 1, 0)

    plsc.subcore_barrier()

    @pl.when(c == 0)
    def _():
        pltpu.sync_copy(agg_spmem.at[pl.ds(s * RPT, RPT)],
                        agg0_hbm.at[pl.ds(s * RPT, RPT)])

    @pl.when(c == 1)
    def _():
        pltpu.sync_copy(agg_spmem.at[pl.ds(s * RPT, RPT)],
                        agg1_hbm.at[pl.ds(s * RPT, RPT)])


def _tc_matmul(xp, W):
    def body(x_ref, w_ref, y_ref):
        y_ref[...] = jnp.dot(x_ref[...], w_ref[...],
                             preferred_element_type=jnp.float32)

    return pl.pallas_call(
        body,
        grid=(N_PAD // RB,),
        in_specs=[
            pl.BlockSpec((RB, D), lambda i: (i, 0)),
            pl.BlockSpec((D, D), lambda i: (0, 0)),
        ],
        out_specs=pl.BlockSpec((RB, D), lambda i: (i, 0)),
        out_shape=jax.ShapeDtypeStruct((N_PAD, D), jnp.float32),
    )(xp, W)


def _tc_scale(y0, deg_src):
    def body(y_ref, deg_ref, o_ref):
        scale = lax.rsqrt(jnp.maximum(deg_ref[:, 0:1], 1.0))
        o_ref[...] = y_ref[...] * scale

    return pl.pallas_call(
        body,
        grid=(N_PAD // RB,),
        in_specs=[
            pl.BlockSpec((RB, D), lambda i: (i, 0)),
            pl.BlockSpec((RB, 8), lambda i: (i, 0)),
        ],
        out_specs=pl.BlockSpec((RB, D), lambda i: (i, 0)),
        out_shape=jax.ShapeDtypeStruct((N_PAD, D), jnp.float32),
    )(y0, deg_src)


def _tc_post(agg0, agg1, deg_dst, b2):
    def body(a0_ref, a1_ref, deg_ref, b_ref, o_ref):
        scale = lax.rsqrt(jnp.maximum(deg_ref[:, 0:1], 1.0))
        o_ref[...] = (a0_ref[...] + a1_ref[...]) * scale + b_ref[0:1, :]

    rb = 1000
    return pl.pallas_call(
        body,
        grid=(N // rb,),
        in_specs=[
            pl.BlockSpec((rb, D), lambda i: (i, 0)),
            pl.BlockSpec((rb, D), lambda i: (i, 0)),
            pl.BlockSpec((rb, 8), lambda i: (i, 0)),
            pl.BlockSpec((1, D), lambda i: (0, 0)),
        ],
        out_specs=pl.BlockSpec((rb, D), lambda i: (i, 0)),
        out_shape=jax.ShapeDtypeStruct((N, D), jnp.float32),
    )(agg0, agg1, deg_dst, b2)


def kernel(x, edge_index, W, b):
    ep = jnp.pad(edge_index, ((0, 0), (0, 2 * CPS * CH - E)),
                 constant_values=N)
    e_all = ep.reshape(4 * CPS, CH)
    dch = (2 * CPS * CH - E) // CH
    spread = (N + (jnp.arange(dch * CH, dtype=jnp.int32) % (N_PAD - N))
              ).reshape(dch, CH)
    e_all = e_all.at[2 * CPS - dch:2 * CPS].set(spread)
    e_all = e_all.at[4 * CPS - dch:].set(spread)
    xp = jnp.pad(x, ((0, N_PAD - N), (0, 0)))

    deg_src, deg_dst = _deg_kernel(e_all, jnp.ones((CH, 8), jnp.float32),
                                   jnp.zeros((N_PAD, 8), jnp.float32))
    y0 = _tc_matmul(xp, W)
    y = _tc_scale(y0, deg_src)
    agg0, agg1 = _agg_kernel(y, e_all)
    return _tc_post(agg0, agg1, deg_dst, b.reshape(1, D))
